# baseline (speedup 1.0000x reference)
import functools

import jax
import jax.numpy as jnp
from jax import lax
from jax.experimental import pallas as pl
from jax.experimental.pallas import tpu as pltpu
from jax.experimental.pallas import tpu_sc as plsc

D = 32
K = 8192
EPS = 1e-05
TT = 1024
KC = 2048


def _ln_argmin_body(x_ref, ct_ref, n_ref, sel_ref, coln_ref):
    @pl.when(pl.program_id(0) == 0)
    def _():
        ct = ct_ref[...]
        coln_ref[...] = jnp.sum(ct * ct, axis=0, keepdims=True)

    x = x_ref[...]
    mean = jnp.mean(x, axis=-1, keepdims=True)
    diff = x - mean
    var = jnp.mean(jnp.square(diff), axis=-1, keepdims=True)
    n_ref[...] = diff / jnp.sqrt(var + EPS)

    rown = jnp.sum(x * x, axis=-1, keepdims=True)

    def dot(j):
        return lax.dot_general(x, ct_ref[:, pl.ds(j * KC, KC)],
                               (((1,), (0,)), ((), ())),
                               preferred_element_type=jnp.float32)

    nj = K // KC
    s = dot(0)
    bmin = jnp.full((TT, 1), jnp.inf, jnp.float32)
    bidx = jnp.zeros((TT, 1), jnp.int32)
    for j in range(nj):
        s_next = dot(j + 1) if j + 1 < nj else None
        coln = coln_ref[:, pl.ds(j * KC, KC)]
        d2 = jnp.maximum((rown - 2.0 * s) + coln, 0.0)
        m2 = jnp.min(d2, axis=-1, keepdims=True)
        tidx = jnp.argmin(d2, axis=-1).reshape(TT, 1) + j * KC
        tmin = jnp.sqrt(m2)
        upd = tmin < bmin
        nmin = jnp.where(upd, tmin, bmin)
        bmin = nmin.astype(jnp.bfloat16).astype(jnp.float32)
        bidx = jnp.where(upd, tidx, bidx)
        s = s_next
    sel_ref[...] = bidx


def _ln_argmin(flat, centroids_t, interpret=False):
    nt = flat.shape[0] // TT
    return pl.pallas_call(
        _ln_argmin_body,
        grid=(nt,),
        in_specs=[
            pl.BlockSpec((TT, D), lambda i: (i, 0)),
            pl.BlockSpec((D, K), lambda i: (0, 0)),
        ],
        out_specs=[
            pl.BlockSpec((TT, D), lambda i: (i, 0)),
            pl.BlockSpec((TT, 1), lambda i: (i, 0)),
        ],
        out_shape=[
            jax.ShapeDtypeStruct((flat.shape[0], D), jnp.float32),
            jax.ShapeDtypeStruct((flat.shape[0], 1), jnp.int32),
        ],
        scratch_shapes=[pltpu.VMEM((1, K), jnp.float32)],
        interpret=interpret,
    )(flat, centroids_t)


N_TOK = 8192
NW = 32
BPW = N_TOK // NW


def _sc_affine_body(n_hbm, sel_hbm, w_hbm, b_hbm, out_hbm,
                    idx_v, w_v, b_v, n_v, o_v, sem_w, sem_b):
    wid = lax.axis_index("s") * 2 + lax.axis_index("c")
    base = wid * BPW
    pltpu.sync_copy(sel_hbm.at[pl.ds(base, BPW)], idx_v)
    cw = pltpu.async_copy(w_hbm.at[idx_v], w_v, sem_w)
    cb = pltpu.async_copy(b_hbm.at[idx_v], b_v, sem_b)
    pltpu.sync_copy(n_hbm.at[pl.ds(base, BPW), :], n_v)
    cw.wait()
    cb.wait()

    def row(r, _):
        for h in (0, 16):
            o_v[r, pl.ds(h, 16)] = (n_v[r, pl.ds(h, 16)] * w_v[r, pl.ds(h, 16)]
                                    + b_v[r, pl.ds(h, 16)])
        return 0

    lax.fori_loop(0, BPW, row, 0)
    pltpu.sync_copy(o_v, out_hbm.at[pl.ds(base, BPW), :])


def _sc_affine(normalized, sel, weights, biases):
    mesh = plsc.VectorSubcoreMesh(core_axis_name="c", subcore_axis_name="s")
    kfn = pl.kernel(
        _sc_affine_body,
        mesh=mesh,
        out_type=jax.ShapeDtypeStruct((N_TOK, D), jnp.float32),
        scratch_types=[
            pltpu.VMEM((BPW,), jnp.int32),
            pltpu.VMEM((BPW, D), jnp.float32),
            pltpu.VMEM((BPW, D), jnp.float32),
            pltpu.VMEM((BPW, D), jnp.float32),
            pltpu.VMEM((BPW, D), jnp.float32),
            pltpu.SemaphoreType.DMA,
            pltpu.SemaphoreType.DMA,
        ],
        compiler_params=pltpu.CompilerParams(use_tc_tiling_on_sc=False),
    )
    return kfn(normalized, sel, weights, biases)


def kernel(input, weights, biases, centroids):
    b, t, d = input.shape
    flat = input.reshape(-1, d)
    normalized, sel2d = _ln_argmin(flat, centroids.T)
    sel = sel2d.reshape(-1)
    out = _sc_affine(normalized, sel, weights, biases)
    return out.reshape(b, t, d), sel.reshape(b, t)


# speedup vs baseline: 1.0353x; 1.0353x over previous
import functools

import jax
import jax.numpy as jnp
from jax import lax
from jax.experimental import pallas as pl
from jax.experimental.pallas import tpu as pltpu
from jax.experimental.pallas import tpu_sc as plsc

D = 32
K = 8192
EPS = 1e-05
TT = 512
KC = 2048


def _ln_argmin_body(x_ref, ct_ref, n_ref, sel_ref, coln_ref):
    @pl.when(pl.program_id(0) == 0)
    def _():
        ct = ct_ref[...]
        coln_ref[...] = jnp.sum(ct * ct, axis=0, keepdims=True)

    x = x_ref[...]
    mean = jnp.mean(x, axis=-1, keepdims=True)
    diff = x - mean
    var = jnp.mean(jnp.square(diff), axis=-1, keepdims=True)
    n_ref[...] = diff / jnp.sqrt(var + EPS)

    rown = jnp.sum(x * x, axis=-1, keepdims=True)

    def dot(j):
        return lax.dot_general(x, ct_ref[:, pl.ds(j * KC, KC)],
                               (((1,), (0,)), ((), ())),
                               preferred_element_type=jnp.float32)

    nj = K // KC
    s = dot(0)
    bmin = jnp.full((TT, 1), jnp.inf, jnp.float32)
    bidx = jnp.zeros((TT, 1), jnp.int32)
    for j in range(nj):
        s_next = dot(j + 1) if j + 1 < nj else None
        coln = coln_ref[:, pl.ds(j * KC, KC)]
        d2 = jnp.maximum((rown - 2.0 * s) + coln, 0.0)
        m2 = jnp.min(d2, axis=-1, keepdims=True)
        tidx = jnp.argmin(d2, axis=-1).reshape(TT, 1) + j * KC
        tmin = jnp.sqrt(m2)
        upd = tmin < bmin
        nmin = jnp.where(upd, tmin, bmin)
        bmin = nmin.astype(jnp.bfloat16).astype(jnp.float32)
        bidx = jnp.where(upd, tidx, bidx)
        s = s_next
    sel_ref[...] = bidx


def _ln_argmin(flat, centroids_t, interpret=False):
    nt = flat.shape[0] // TT
    return pl.pallas_call(
        _ln_argmin_body,
        grid=(nt,),
        in_specs=[
            pl.BlockSpec((TT, D), lambda i: (i, 0)),
            pl.BlockSpec((D, K), lambda i: (0, 0)),
        ],
        out_specs=[
            pl.BlockSpec((TT, D), lambda i: (i, 0)),
            pl.BlockSpec((TT, 1), lambda i: (i, 0)),
        ],
        out_shape=[
            jax.ShapeDtypeStruct((flat.shape[0], D), jnp.float32),
            jax.ShapeDtypeStruct((flat.shape[0], 1), jnp.int32),
        ],
        scratch_shapes=[pltpu.VMEM((1, K), jnp.float32)],
        interpret=interpret,
    )(flat, centroids_t)


N_TOK = 8192
NW = 32
BPW = N_TOK // NW


def _sc_affine_body(n_hbm, sel_hbm, w_hbm, b_hbm, out_hbm,
                    idx_v, w_v, b_v, n_v, o_v, sem_w, sem_b):
    wid = lax.axis_index("s") * 2 + lax.axis_index("c")
    base = wid * BPW
    pltpu.sync_copy(sel_hbm.at[pl.ds(base, BPW)], idx_v)
    cw = pltpu.async_copy(w_hbm.at[idx_v], w_v, sem_w)
    cb = pltpu.async_copy(b_hbm.at[idx_v], b_v, sem_b)
    pltpu.sync_copy(n_hbm.at[pl.ds(base, BPW), :], n_v)
    cw.wait()
    cb.wait()

    def row(r, _):
        for h in (0, 16):
            o_v[r, pl.ds(h, 16)] = (n_v[r, pl.ds(h, 16)] * w_v[r, pl.ds(h, 16)]
                                    + b_v[r, pl.ds(h, 16)])
        return 0

    lax.fori_loop(0, BPW, row, 0)
    pltpu.sync_copy(o_v, out_hbm.at[pl.ds(base, BPW), :])


def _sc_affine(normalized, sel, weights, biases):
    mesh = plsc.VectorSubcoreMesh(core_axis_name="c", subcore_axis_name="s")
    kfn = pl.kernel(
        _sc_affine_body,
        mesh=mesh,
        out_type=jax.ShapeDtypeStruct((N_TOK, D), jnp.float32),
        scratch_types=[
            pltpu.VMEM((BPW,), jnp.int32),
            pltpu.VMEM((BPW, D), jnp.float32),
            pltpu.VMEM((BPW, D), jnp.float32),
            pltpu.VMEM((BPW, D), jnp.float32),
            pltpu.VMEM((BPW, D), jnp.float32),
            pltpu.SemaphoreType.DMA,
            pltpu.SemaphoreType.DMA,
        ],
        compiler_params=pltpu.CompilerParams(use_tc_tiling_on_sc=False),
    )
    return kfn(normalized, sel, weights, biases)


def kernel(input, weights, biases, centroids):
    b, t, d = input.shape
    flat = input.reshape(-1, d)
    normalized, sel2d = _ln_argmin(flat, centroids.T)
    sel = sel2d.reshape(-1)
    out = _sc_affine(normalized, sel, weights, biases)
    return out.reshape(b, t, d), sel.reshape(b, t)
